# baseline (speedup 1.0000x reference)
import functools

import jax
import jax.numpy as jnp
from jax.experimental import pallas as pl
from jax.experimental.pallas import tpu as pltpu


def _mask_mul_body(s_ref, r_ref, x_ref, out_ref, mask_ref, *, C):
    s_row = s_ref[0]
    s_col = s_row.reshape(C, 1)
    rate = r_ref[0, 0, 0]
    k = jnp.clip(jnp.round(rate * C), 1.0, float(C))

    j_idx = jax.lax.broadcasted_iota(jnp.int32, (C, C), 0)
    c_idx = jax.lax.broadcasted_iota(jnp.int32, (C, C), 1)
    beats = (s_col > s_row) | ((s_col == s_row) & (j_idx < c_idx))
    ranks = jnp.sum(beats.astype(jnp.float32), axis=0, keepdims=True)

    mask_row = (ranks < k).astype(x_ref.dtype)
    mask_ref[0] = mask_row
    out_ref[0] = x_ref[0] * mask_row.reshape(C, 1)


def kernel(x, channel_scores, rate_ratio):
    B, C, H, W = x.shape
    HW = H * W
    rate = jnp.reshape(jnp.asarray(rate_ratio, dtype=x.dtype), (-1,))
    if rate.shape[0] == 1:
        rate = jnp.broadcast_to(rate, (B,))
    active_channels = jnp.clip(jnp.round(rate * C).astype(jnp.int64), 1, C)

    x3 = x.reshape(B, C, HW)
    scores3 = channel_scores.astype(x.dtype).reshape(B, 1, C)
    rate3 = rate.reshape(B, 1, 1)

    masked3, mask3 = pl.pallas_call(
        functools.partial(_mask_mul_body, C=C),
        grid=(B,),
        in_specs=[
            pl.BlockSpec((1, 1, C), lambda b: (b, 0, 0)),
            pl.BlockSpec((1, 1, 1), lambda b: (b, 0, 0)),
            pl.BlockSpec((1, C, HW), lambda b: (b, 0, 0)),
        ],
        out_specs=[
            pl.BlockSpec((1, C, HW), lambda b: (b, 0, 0)),
            pl.BlockSpec((1, 1, C), lambda b: (b, 0, 0)),
        ],
        out_shape=[
            jax.ShapeDtypeStruct((B, C, HW), x.dtype),
            jax.ShapeDtypeStruct((B, 1, C), x.dtype),
        ],
        compiler_params=pltpu.CompilerParams(
            dimension_semantics=("arbitrary",),
        ),
    )(scores3, rate3, x3)

    masked = masked3.reshape(B, C, H, W)
    mask = mask3.reshape(B, C)
    spatial_mask = mask[:, :, None, None]
    return (masked, mask, spatial_mask, active_channels, rate)
